# baseline (speedup 1.0000x reference)
import functools

import jax
import jax.numpy as jnp
from jax import lax
from jax.experimental import pallas as pl
from jax.experimental.pallas import tpu as pltpu
from jax.experimental.pallas import tpu_sc as plsc

_C = 16
_NBUF = 4


@functools.lru_cache(maxsize=None)
def _make_kernel(B: int, D: int, V: int):
    info = plsc.get_sparse_core_info()
    nc, ns = info.num_cores, info.num_subcores
    nw = nc * ns
    assert B % (8 * nw) == 0
    b_per_w = B // nw
    n_chunks = b_per_w // _C
    assert b_per_w % _C == 0 and n_chunks % _NBUF == 0 and n_chunks >= 2 * _NBUF
    n_groups = n_chunks // _NBUF
    mesh = plsc.VectorSubcoreMesh(core_axis_name="c", subcore_axis_name="s")

    @functools.partial(
        pl.kernel,
        mesh=mesh,
        out_type=jax.ShapeDtypeStruct((B, D), jnp.float32),
        scratch_types=[
            pltpu.VMEM((b_per_w,), jnp.int32),
            *([pltpu.VMEM((_C, D), jnp.float32)] * _NBUF),
            *([pltpu.SemaphoreType.DMA] * (2 * _NBUF)),
        ],
    )
    def k(table_hbm, idx_hbm, out_hbm, idx_v, *rest):
        bufs = rest[:_NBUF]
        gsem = rest[_NBUF:2 * _NBUF]
        osem = rest[2 * _NBUF:]
        wid = lax.axis_index("s") * nc + lax.axis_index("c")
        base = wid * b_per_w
        pltpu.sync_copy(idx_hbm.at[pl.ds(base, b_per_w)], idx_v)

        def start_gather(i, b):
            vec = idx_v[pl.ds(i * _C, _C)]
            vec = jnp.minimum(jnp.maximum(vec, 0), V - 1)
            pltpu.async_copy(table_hbm.at[vec], bufs[b], gsem[b])

        def wait_gather(b):
            pltpu.make_async_copy(
                table_hbm.at[idx_v.at[pl.ds(0, _C)]], bufs[b], gsem[b]
            ).wait()

        def start_out(i, b):
            pltpu.async_copy(bufs[b], out_hbm.at[pl.ds(base + i * _C, _C)], osem[b])

        def wait_out(b):
            pltpu.make_async_copy(
                bufs[b], out_hbm.at[pl.ds(base, _C)], osem[b]
            ).wait()

        start_gather(0, 0)
        start_gather(1, 1)
        start_gather(2, 2)
        wait_gather(0)
        start_out(0, 0)
        start_gather(3, 3)
        wait_gather(1)
        start_out(1, 1)

        def group(j, carry):
            for b in range(_NBUF):
                i = j * _NBUF + b
                bo = (b + 2) % _NBUF
                wait_out(b)
                start_gather(i, b)
                wait_gather(bo)
                start_out(i - 2, bo)
            return carry

        lax.fori_loop(1, n_groups, group, 0)

        last = n_chunks - 4
        wait_gather(2)
        start_out(last + 2, 2)
        wait_gather(3)
        start_out(last + 3, 3)
        for b in range(_NBUF):
            wait_out(b)

    return k


@functools.lru_cache(maxsize=None)
def _make_tc_kernel(R: int, V: int, D: int):
    br = 1024
    assert R % br == 0 and D % 128 == 0
    sl = D // 128

    def body(idx_ref, table_ref, out_ref):
        g = pl.program_id(0)

        def row(i, carry):
            r = idx_ref[g * br + i]
            r = jnp.minimum(jnp.maximum(r, 0), V - 1)
            out_ref[pl.ds(i, 1)] = table_ref[pl.ds(r, 1)]
            return carry

        lax.fori_loop(0, br, row, 0, unroll=8)

    return pl.pallas_call(
        body,
        grid_spec=pltpu.PrefetchScalarGridSpec(
            num_scalar_prefetch=1,
            grid=(R // br,),
            in_specs=[pl.BlockSpec((V, sl, 128), lambda g, idx_ref: (0, 0, 0))],
            out_specs=pl.BlockSpec((br, sl, 128), lambda g, idx_ref: (g, 0, 0)),
        ),
        out_shape=jax.ShapeDtypeStruct((R, sl, 128), jnp.float32),
    )


_TC_ROWS = 8192


def kernel(pos_idxs, pos_emb):
    bsz, seq = pos_idxs.shape
    v, d = pos_emb.shape
    n = bsz * seq
    idx = pos_idxs.reshape(n).astype(jnp.int32)
    n_sc = n - _TC_ROWS
    out_sc = _make_kernel(n_sc, d, v)(pos_emb, idx[:n_sc])
    out_tc = _make_tc_kernel(_TC_ROWS, v, d)(
        idx[n_sc:], pos_emb.reshape(v, d // 128, 128)
    )
    out = jnp.concatenate([out_sc, out_tc.reshape(_TC_ROWS, d)], axis=0)
    return out.reshape(bsz, seq, d)


# speedup vs baseline: 2.3738x; 2.3738x over previous
import functools

import jax
import jax.numpy as jnp
from jax import lax
from jax.experimental import pallas as pl
from jax.experimental.pallas import tpu as pltpu
from jax.experimental.pallas import tpu_sc as plsc

_C = 16
_NBUF = 4


@functools.lru_cache(maxsize=None)
def _make_kernel(B: int, D: int, V: int):
    info = plsc.get_sparse_core_info()
    nc, ns = info.num_cores, info.num_subcores
    nw = nc * ns
    assert B % (8 * nw) == 0
    b_per_w = B // nw
    n_chunks = b_per_w // _C
    assert b_per_w % _C == 0 and n_chunks % _NBUF == 0 and n_chunks >= 2 * _NBUF
    n_groups = n_chunks // _NBUF
    mesh = plsc.VectorSubcoreMesh(core_axis_name="c", subcore_axis_name="s")

    @functools.partial(
        pl.kernel,
        mesh=mesh,
        out_type=jax.ShapeDtypeStruct((B, D), jnp.float32),
        scratch_types=[
            pltpu.VMEM((b_per_w,), jnp.int32),
            *([pltpu.VMEM((_C, D), jnp.float32)] * _NBUF),
            *([pltpu.SemaphoreType.DMA] * (2 * _NBUF)),
        ],
    )
    def k(table_hbm, idx_hbm, out_hbm, idx_v, *rest):
        bufs = rest[:_NBUF]
        gsem = rest[_NBUF:2 * _NBUF]
        osem = rest[2 * _NBUF:]
        wid = lax.axis_index("s") * nc + lax.axis_index("c")
        base = wid * b_per_w
        pltpu.sync_copy(idx_hbm.at[pl.ds(base, b_per_w)], idx_v)

        def start_gather(i, b):
            vec = idx_v[pl.ds(i * _C, _C)]
            vec = jnp.minimum(jnp.maximum(vec, 0), V - 1)
            pltpu.async_copy(table_hbm.at[vec], bufs[b], gsem[b])

        def wait_gather(b):
            pltpu.make_async_copy(
                table_hbm.at[idx_v.at[pl.ds(0, _C)]], bufs[b], gsem[b]
            ).wait()

        def start_out(i, b):
            pltpu.async_copy(bufs[b], out_hbm.at[pl.ds(base + i * _C, _C)], osem[b])

        def wait_out(b):
            pltpu.make_async_copy(
                bufs[b], out_hbm.at[pl.ds(base, _C)], osem[b]
            ).wait()

        start_gather(0, 0)
        start_gather(1, 1)
        start_gather(2, 2)
        wait_gather(0)
        start_out(0, 0)
        start_gather(3, 3)
        wait_gather(1)
        start_out(1, 1)

        def group(j, carry):
            for b in range(_NBUF):
                i = j * _NBUF + b
                bo = (b + 2) % _NBUF
                wait_out(b)
                start_gather(i, b)
                wait_gather(bo)
                start_out(i - 2, bo)
            return carry

        lax.fori_loop(1, n_groups, group, 0)

        last = n_chunks - 4
        wait_gather(2)
        start_out(last + 2, 2)
        wait_gather(3)
        start_out(last + 3, 3)
        for b in range(_NBUF):
            wait_out(b)

    return k


def kernel(pos_idxs, pos_emb):
    bsz, seq = pos_idxs.shape
    v, d = pos_emb.shape
    idx = pos_idxs.reshape(bsz * seq).astype(jnp.int32)
    out = _make_kernel(bsz * seq, d, v)(pos_emb, idx)
    return out.reshape(bsz, seq, d)
